# baseline (speedup 1.0000x reference)
import functools

import jax
import jax.numpy as jnp
from jax import lax
from jax.experimental import pallas as pl
from jax.experimental.pallas import tpu as pltpu
from jax.experimental.pallas import tpu_sc as plsc

N, L, D = 32, 1024, 256
B = N * L
LANES = 16
NUM_WORKERS = 32
B_PER_W = B // NUM_WORKERS
H = B_PER_W // 2
CHUNK = 64
N_CHUNKS = H // CHUNK


def _make_lookup():
  mesh = plsc.VectorSubcoreMesh(core_axis_name="c", subcore_axis_name="s")

  @functools.partial(
      pl.kernel,
      mesh=mesh,
      out_type=jax.ShapeDtypeStruct((B, D), jnp.float32),
      scratch_types=[
          pltpu.VMEM((B_PER_W,), jnp.int32),
          pltpu.VMEM((B_PER_W,), jnp.int32),
          pltpu.VMEM((CHUNK, 2, D), jnp.float32),
          pltpu.VMEM((CHUNK, 2, D), jnp.float32),
          pltpu.VMEM((CHUNK, 2, D), jnp.float32),
          pltpu.SemaphoreType.DMA,
          pltpu.SemaphoreType.DMA,
          pltpu.SemaphoreType.DMA,
          pltpu.SemaphoreType.DMA,
          pltpu.SemaphoreType.DMA,
          pltpu.SemaphoreType.DMA,
          pltpu.SemaphoreType.DMA,
          pltpu.SemaphoreType.DMA,
          pltpu.SemaphoreType.DMA,
      ],
  )
  def lookup(aa_hbm, mask_hbm, table_hbm, out_hbm, idx_v, mask_v, rows_a,
             rows_b, rows_c, sg_a, sg_b, sg_c, ss_a, ss_b, ss_c, st_a, st_b,
             st_c):
    wid = lax.axis_index("s") * 2 + lax.axis_index("c")
    base = wid * B_PER_W
    pltpu.sync_copy(aa_hbm.at[pl.ds(base, B_PER_W)], idx_v)
    pltpu.sync_copy(mask_hbm.at[pl.ds(base, B_PER_W)], mask_v)
    tab_base = wid * 64
    for j in range(H // LANES):
      sl = pl.ds(j * LANES, LANES)
      s2 = pl.ds(H + j * LANES, LANES)
      e = jnp.where(mask_v[sl] == 0, jnp.int32(7), idx_v[sl])
      o = jnp.where(mask_v[s2] == 0, jnp.int32(7), idx_v[s2])
      idx_v[sl] = e * 8 + o + tab_base
    rows = (rows_a, rows_b, rows_c)
    sg = (sg_a, sg_b, sg_c)
    ss = (ss_a, ss_b, ss_c)
    st = (st_a, st_b, st_c)
    nbuf = len(rows)
    g = [None] * N_CHUNKS
    s0 = [None] * N_CHUNKS
    s1 = [None] * N_CHUNKS

    def scatter(c):
      bb = c % nbuf
      s0[c] = pltpu.async_copy(
          rows[bb].at[:, 0],
          out_hbm.at[pl.ds(base + c * CHUNK, CHUNK)], ss[bb]
      )
      s1[c] = pltpu.async_copy(
          rows[bb].at[:, 1],
          out_hbm.at[pl.ds(base + H + c * CHUNK, CHUNK)], st[bb]
      )

    for c in range(N_CHUNKS):
      b = c % nbuf
      if c >= nbuf:
        s0[c - nbuf].wait()
        s1[c - nbuf].wait()
      g[c] = pltpu.async_copy(
          table_hbm.at[idx_v.at[pl.ds(c * CHUNK, CHUNK)]], rows[b], sg[b]
      )
      if c >= 1:
        g[c - 1].wait()
        scatter(c - 1)
    last = N_CHUNKS - 1
    g[last].wait()
    scatter(last)
    for c in range(max(0, N_CHUNKS - nbuf), N_CHUNKS):
      s0[c].wait()
      s1[c].wait()

  return lookup


_lookup = _make_lookup()


def kernel(aa, res_nb, chain_nb, pos_atoms, mask_atoms, fragment_type, emb_table):
  aa_flat = aa.reshape(B).astype(jnp.int32)
  mask_flat = mask_atoms[:, :, 0].reshape(B).astype(jnp.int32)
  table_ext = jnp.concatenate(
      [emb_table.astype(jnp.float32), jnp.zeros((1, D), jnp.float32)], axis=0
  )
  pair_tab = jnp.stack(
      [jnp.repeat(table_ext, 8, axis=0), jnp.tile(table_ext, (8, 1))], axis=1
  )
  table_rep = jnp.tile(pair_tab, (NUM_WORKERS, 1, 1))
  out = _lookup(aa_flat, mask_flat, table_rep)
  return out.reshape(N, L, D)


# speedup vs baseline: 1.0483x; 1.0483x over previous
import functools

import jax
import jax.numpy as jnp
from jax import lax
from jax.experimental import pallas as pl
from jax.experimental.pallas import tpu as pltpu
from jax.experimental.pallas import tpu_sc as plsc

N, L, D = 32, 1024, 256
B = N * L
LANES = 16
NUM_WORKERS = 32
B_PER_W = B // NUM_WORKERS
CHUNK = 128
N_CHUNKS = B_PER_W // CHUNK
REPS = 16


def _make_lookup():
  mesh = plsc.VectorSubcoreMesh(core_axis_name="c", subcore_axis_name="s")

  @functools.partial(
      pl.kernel,
      mesh=mesh,
      out_type=jax.ShapeDtypeStruct((B, D), jnp.float32),
      scratch_types=[
          pltpu.VMEM((B_PER_W,), jnp.int32),
          pltpu.VMEM((B_PER_W,), jnp.int32),
          pltpu.VMEM((CHUNK, D), jnp.float32),
          pltpu.VMEM((CHUNK, D), jnp.float32),
          pltpu.VMEM((CHUNK, D), jnp.float32),
          pltpu.SemaphoreType.DMA,
          pltpu.SemaphoreType.DMA,
          pltpu.SemaphoreType.DMA,
          pltpu.SemaphoreType.DMA,
          pltpu.SemaphoreType.DMA,
          pltpu.SemaphoreType.DMA,
      ],
  )
  def lookup(aa_hbm, mask_hbm, table_hbm, out_hbm, idx_v, mask_v, rows_a,
             rows_b, rows_c, sg_a, sg_b, sg_c, ss_a, ss_b, ss_c):
    wid = lax.axis_index("s") * 2 + lax.axis_index("c")
    base = wid * B_PER_W
    pltpu.sync_copy(aa_hbm.at[pl.ds(base, B_PER_W)], idx_v)
    pltpu.sync_copy(mask_hbm.at[pl.ds(base, B_PER_W)], mask_v)
    tab_base = wid * (8 * REPS)
    for j in range(B_PER_W // LANES):
      sl = pl.ds(j * LANES, LANES)
      a = idx_v[sl]
      m = mask_v[sl]
      idx_v[sl] = jnp.where(m == 0, jnp.int32(7), a) + (
          tab_base + (j % REPS) * 8
      )
    rows = (rows_a, rows_b, rows_c)
    sg = (sg_a, sg_b, sg_c)
    ss = (ss_a, ss_b, ss_c)
    nbuf = len(rows)
    g = [None] * N_CHUNKS
    s = [None] * N_CHUNKS
    for c in range(N_CHUNKS):
      b = c % nbuf
      if c >= nbuf:
        s[c - nbuf].wait()
      g[c] = pltpu.async_copy(
          table_hbm.at[idx_v.at[pl.ds(c * CHUNK, CHUNK)]], rows[b], sg[b]
      )
      if c >= 1:
        p = c - 1
        g[p].wait()
        s[p] = pltpu.async_copy(
            rows[p % nbuf], out_hbm.at[pl.ds(base + p * CHUNK, CHUNK)],
            ss[p % nbuf]
        )
    last = N_CHUNKS - 1
    g[last].wait()
    s[last] = pltpu.async_copy(
        rows[last % nbuf], out_hbm.at[pl.ds(base + last * CHUNK, CHUNK)],
        ss[last % nbuf]
    )
    for c in range(max(0, N_CHUNKS - nbuf), N_CHUNKS):
      s[c].wait()

  return lookup


_lookup = _make_lookup()


def kernel(aa, res_nb, chain_nb, pos_atoms, mask_atoms, fragment_type, emb_table):
  aa_flat = aa.reshape(B).astype(jnp.int32)
  mask_flat = mask_atoms[:, :, 0].reshape(B).astype(jnp.int32)
  table_ext = jnp.concatenate(
      [emb_table.astype(jnp.float32), jnp.zeros((1, D), jnp.float32)], axis=0
  )
  table_rep = jnp.tile(table_ext, (NUM_WORKERS * REPS, 1))
  out = _lookup(aa_flat, mask_flat, table_rep)
  return out.reshape(N, L, D)
